# baseline (speedup 1.0000x reference)
import functools

import jax
import jax.numpy as jnp
from jax import lax
from jax.experimental import pallas as pl
from jax.experimental.pallas import tpu as pltpu
from jax.experimental.pallas import tpu_sc as plsc

N_NODES = 10000
N_EDGES = 160000
D_IN = 256
W_HID = 512
D_OUT = 256

NC = 2
NS = 16
DH = 128

K = 96
NB = 4
NCHUNK = 108
EPT = NCHUNK * K
E_PAD = EPT * NS
NPT = 624
NPT_LAST = N_NODES - (NS - 1) * NPT
ACC_ROWS = N_NODES + 16


def _sc_aggregate(xa, xb, src, dst):
    mesh = plsc.VectorSubcoreMesh(
        core_axis_name="c", subcore_axis_name="s", num_cores=NC, num_subcores=NS
    )

    @functools.partial(
        pl.kernel,
        out_type=(
            jax.ShapeDtypeStruct((N_NODES, DH), jnp.float32),
            jax.ShapeDtypeStruct((N_NODES, DH), jnp.float32),
        ),
        mesh=mesh,
        scratch_types=[
            [pltpu.VMEM((K,), jnp.int32) for _ in range(NB)],
            [pltpu.VMEM((K,), jnp.int32) for _ in range(NB)],
            [pltpu.VMEM((K, DH), jnp.float32) for _ in range(NB)],
            pltpu.VMEM_SHARED((ACC_ROWS, DH), jnp.float32),
            pltpu.SemaphoreType.DMA,
            pltpu.SemaphoreType.DMA,
            pltpu.SemaphoreType.DMA,
        ],
    )
    def body(xa_hbm, xb_hbm, src_hbm, dst_hbm, outa_hbm, outb_hbm,
             sidx, didx, rows, acc, sem_g, sem_si, sem_di):
        c = lax.axis_index("c")
        s = lax.axis_index("s")
        base = s * EPT

        def sidx_copy(j, b):
            off = pl.multiple_of(base + j * K, K)
            pltpu.async_copy(src_hbm.at[pl.ds(off, K)], sidx[b], sem_si)

        def didx_copy(j, b):
            off = pl.multiple_of(base + j * K, K)
            pltpu.async_copy(dst_hbm.at[pl.ds(off, K)], didx[b], sem_di)

        def wait_sidx(b):
            pltpu.make_async_copy(src_hbm.at[pl.ds(0, K)], sidx[b], sem_si).wait()

        def wait_didx(b):
            pltpu.make_async_copy(dst_hbm.at[pl.ds(0, K)], didx[b], sem_di).wait()

        def gather(x_hbm, b):
            return pltpu.async_copy(x_hbm.at[sidx[b]], rows[b], sem_g)

        def wait_gather(x_hbm, b):
            pltpu.make_async_copy(x_hbm.at[sidx[b]], rows[b], sem_g).wait()

        def prime(x_hbm):
            for b in range(NB):
                sidx_copy(b, b)
                didx_copy(b, b)
            for b in range(NB):
                wait_sidx(b)
                gather(x_hbm, b)

        @pl.when(c == 0)
        def _():
            prime(xa_hbm)

        @pl.when(c == 1)
        def _():
            prime(xb_hbm)

        def seed(x_hbm):
            @pl.when(s < NS - 1)
            def _():
                pltpu.sync_copy(x_hbm.at[pl.ds(s * NPT, NPT)],
                                acc.at[pl.ds(s * NPT, NPT)])

            @pl.when(s == NS - 1)
            def _():
                pltpu.sync_copy(x_hbm.at[pl.ds((NS - 1) * NPT, NPT_LAST)],
                                acc.at[pl.ds((NS - 1) * NPT, NPT_LAST)])

        @pl.when(c == 0)
        def _():
            seed(xa_hbm)

        @pl.when(c == 1)
        def _():
            seed(xb_hbm)

        plsc.subcore_barrier()

        def run(x_hbm):
            def step(j, b):
                wait_gather(x_hbm, b)

                @pl.when(j + NB < NCHUNK)
                def _():
                    sidx_copy(j + NB, b)

                wait_didx(b)
                pltpu.sync_copy(rows[b], acc.at[didx[b]], add=True)

                @pl.when(j + NB < NCHUNK)
                def _():
                    wait_sidx(b)
                    gather(x_hbm, b)
                    didx_copy(j + NB, b)

            def group(p, carry):
                j0 = NB * p
                for b in range(NB):
                    step(j0 + b, b)
                return carry

            lax.fori_loop(0, NCHUNK // NB, group, 0)

        @pl.when(c == 0)
        def _():
            run(xa_hbm)

        @pl.when(c == 1)
        def _():
            run(xb_hbm)

        plsc.subcore_barrier()

        def writeback(out_hbm):
            @pl.when(s < NS - 1)
            def _():
                pltpu.sync_copy(acc.at[pl.ds(s * NPT, NPT)],
                                out_hbm.at[pl.ds(s * NPT, NPT)])

            @pl.when(s == NS - 1)
            def _():
                pltpu.sync_copy(acc.at[pl.ds((NS - 1) * NPT, NPT_LAST)],
                                out_hbm.at[pl.ds((NS - 1) * NPT, NPT_LAST)])

        @pl.when(c == 0)
        def _():
            writeback(outa_hbm)

        @pl.when(c == 1)
        def _():
            writeback(outb_hbm)

    return body(xa, xb, src, dst)


def _mlp_body(ha_ref, hb_ref, w1_ref, b1_ref, w2_ref, b2_ref, o_ref):
    h = jnp.concatenate([ha_ref[...], hb_ref[...]], axis=1)
    z = jnp.dot(h, w1_ref[...], preferred_element_type=jnp.float32) + b1_ref[...]
    z = jnp.maximum(z, 0.0)
    o_ref[...] = (
        jnp.dot(z, w2_ref[...], preferred_element_type=jnp.float32) + b2_ref[...]
    )


def _mlp(ha, hb, W1, b1, W2, b2):
    BN = 1000
    grid = (N_NODES // BN,)
    return pl.pallas_call(
        _mlp_body,
        grid=grid,
        in_specs=[
            pl.BlockSpec((BN, 128), lambda i: (i, 0)),
            pl.BlockSpec((BN, 128), lambda i: (i, 0)),
            pl.BlockSpec((D_IN, W_HID), lambda i: (0, 0)),
            pl.BlockSpec((1, W_HID), lambda i: (0, 0)),
            pl.BlockSpec((W_HID, D_OUT), lambda i: (0, 0)),
            pl.BlockSpec((1, D_OUT), lambda i: (0, 0)),
        ],
        out_specs=pl.BlockSpec((BN, D_OUT), lambda i: (i, 0)),
        out_shape=jax.ShapeDtypeStruct((N_NODES, D_OUT), jnp.float32),
    )(ha, hb, W1, b1.reshape(1, W_HID), W2, b2.reshape(1, D_OUT))


def kernel(x, edge_index, W1, b1, W2, b2):
    src = edge_index[0].astype(jnp.int32)
    dst = edge_index[1].astype(jnp.int32)
    pad = E_PAD - N_EDGES
    src = jnp.concatenate([src, jnp.zeros((pad,), jnp.int32)])
    dst = jnp.concatenate([dst, jnp.full((pad,), N_NODES, jnp.int32)])
    xa = x[:, :DH]
    xb = x[:, DH:2*DH]
    ha, hb = _sc_aggregate(xa, xb, src, dst)
    return _mlp(ha, hb, W1, b1, W2, b2)


# speedup vs baseline: 1.3273x; 1.3273x over previous
import functools

import jax
import jax.numpy as jnp
from jax import lax
from jax.experimental import pallas as pl
from jax.experimental.pallas import tpu as pltpu
from jax.experimental.pallas import tpu_sc as plsc

N_NODES = 10000
N_EDGES = 160000
D_IN = 256
W_HID = 512
D_OUT = 256

NC = 2
NS = 16
DH = 128

K = 64
NB = 5
NCHUNK = 160
EPT = NCHUNK * K
E_PAD = EPT * NS
NPT = 624
NPT_LAST = N_NODES - (NS - 1) * NPT
ACC_ROWS = N_NODES + 16


def _sc_aggregate(xa, xb, src, dst):
    mesh = plsc.VectorSubcoreMesh(
        core_axis_name="c", subcore_axis_name="s", num_cores=NC, num_subcores=NS
    )

    @functools.partial(
        pl.kernel,
        out_type=(
            jax.ShapeDtypeStruct((N_NODES, DH), jnp.float32),
            jax.ShapeDtypeStruct((N_NODES, DH), jnp.float32),
        ),
        mesh=mesh,
        scratch_types=[
            [pltpu.VMEM((K,), jnp.int32) for _ in range(NB)],
            [pltpu.VMEM((K,), jnp.int32) for _ in range(NB)],
            [pltpu.VMEM((K, DH), jnp.float32) for _ in range(NB)],
            pltpu.VMEM_SHARED((ACC_ROWS, DH), jnp.float32),
            pltpu.SemaphoreType.DMA,
            pltpu.SemaphoreType.DMA,
            pltpu.SemaphoreType.DMA,
        ],
    )
    def body(xa_hbm, xb_hbm, src_hbm, dst_hbm, outa_hbm, outb_hbm,
             sidx, didx, rows, acc, sem_g, sem_si, sem_di):
        c = lax.axis_index("c")
        s = lax.axis_index("s")
        base = s * EPT

        def sidx_copy(j, b):
            off = pl.multiple_of(base + j * K, K)
            pltpu.async_copy(src_hbm.at[pl.ds(off, K)], sidx[b], sem_si)

        def didx_copy(j, b):
            off = pl.multiple_of(base + j * K, K)
            pltpu.async_copy(dst_hbm.at[pl.ds(off, K)], didx[b], sem_di)

        def wait_sidx(b):
            pltpu.make_async_copy(src_hbm.at[pl.ds(0, K)], sidx[b], sem_si).wait()

        def wait_didx(b):
            pltpu.make_async_copy(dst_hbm.at[pl.ds(0, K)], didx[b], sem_di).wait()

        def gather(x_hbm, b):
            return pltpu.async_copy(x_hbm.at[sidx[b]], rows[b], sem_g)

        def wait_gather(x_hbm, b):
            pltpu.make_async_copy(x_hbm.at[sidx[b]], rows[b], sem_g).wait()

        def prime(x_hbm):
            for b in range(NB):
                sidx_copy(b, b)
                didx_copy(b, b)
            for b in range(NB):
                wait_sidx(b)
                gather(x_hbm, b)

        @pl.when(c == 0)
        def _():
            prime(xa_hbm)

        @pl.when(c == 1)
        def _():
            prime(xb_hbm)

        def seed(x_hbm):
            @pl.when(s < NS - 1)
            def _():
                pltpu.sync_copy(x_hbm.at[pl.ds(s * NPT, NPT)],
                                acc.at[pl.ds(s * NPT, NPT)])

            @pl.when(s == NS - 1)
            def _():
                pltpu.sync_copy(x_hbm.at[pl.ds((NS - 1) * NPT, NPT_LAST)],
                                acc.at[pl.ds((NS - 1) * NPT, NPT_LAST)])

        @pl.when(c == 0)
        def _():
            seed(xa_hbm)

        @pl.when(c == 1)
        def _():
            seed(xb_hbm)

        plsc.subcore_barrier()

        def run(x_hbm):
            def step(j, b):
                wait_gather(x_hbm, b)

                @pl.when(j + NB < NCHUNK)
                def _():
                    sidx_copy(j + NB, b)

                wait_didx(b)
                pltpu.sync_copy(rows[b], acc.at[didx[b]], add=True)

                @pl.when(j + NB < NCHUNK)
                def _():
                    wait_sidx(b)
                    gather(x_hbm, b)
                    didx_copy(j + NB, b)

            def group(p, carry):
                j0 = NB * p
                for b in range(NB):
                    step(j0 + b, b)
                return carry

            lax.fori_loop(0, NCHUNK // NB, group, 0)

        @pl.when(c == 0)
        def _():
            run(xa_hbm)

        @pl.when(c == 1)
        def _():
            run(xb_hbm)

        plsc.subcore_barrier()

        def writeback(out_hbm):
            @pl.when(s < NS - 1)
            def _():
                pltpu.sync_copy(acc.at[pl.ds(s * NPT, NPT)],
                                out_hbm.at[pl.ds(s * NPT, NPT)])

            @pl.when(s == NS - 1)
            def _():
                pltpu.sync_copy(acc.at[pl.ds((NS - 1) * NPT, NPT_LAST)],
                                out_hbm.at[pl.ds((NS - 1) * NPT, NPT_LAST)])

        @pl.when(c == 0)
        def _():
            writeback(outa_hbm)

        @pl.when(c == 1)
        def _():
            writeback(outb_hbm)

    return body(xa, xb, src, dst)


def _mlp_body(ha_ref, hb_ref, w1_ref, b1_ref, w2_ref, b2_ref, o_ref):
    h = jnp.concatenate([ha_ref[...], hb_ref[...]], axis=1)
    z = jnp.dot(h, w1_ref[...], preferred_element_type=jnp.float32) + b1_ref[...]
    z = jnp.maximum(z, 0.0)
    o_ref[...] = (
        jnp.dot(z, w2_ref[...], preferred_element_type=jnp.float32) + b2_ref[...]
    )


def _mlp(ha, hb, W1, b1, W2, b2):
    BN = 1000
    grid = (N_NODES // BN,)
    return pl.pallas_call(
        _mlp_body,
        grid=grid,
        in_specs=[
            pl.BlockSpec((BN, 128), lambda i: (i, 0)),
            pl.BlockSpec((BN, 128), lambda i: (i, 0)),
            pl.BlockSpec((D_IN, W_HID), lambda i: (0, 0)),
            pl.BlockSpec((1, W_HID), lambda i: (0, 0)),
            pl.BlockSpec((W_HID, D_OUT), lambda i: (0, 0)),
            pl.BlockSpec((1, D_OUT), lambda i: (0, 0)),
        ],
        out_specs=pl.BlockSpec((BN, D_OUT), lambda i: (i, 0)),
        out_shape=jax.ShapeDtypeStruct((N_NODES, D_OUT), jnp.float32),
    )(ha, hb, W1, b1.reshape(1, W_HID), W2, b2.reshape(1, D_OUT))


def kernel(x, edge_index, W1, b1, W2, b2):
    src = edge_index[0].astype(jnp.int32)
    dst = edge_index[1].astype(jnp.int32)
    pad = E_PAD - N_EDGES
    src = jnp.concatenate([src, jnp.zeros((pad,), jnp.int32)])
    dst = jnp.concatenate([dst, jnp.full((pad,), N_NODES, jnp.int32)])
    xa = x[:, :DH]
    xb = x[:, DH:2*DH]
    ha, hb = _sc_aggregate(xa, xb, src, dst)
    return _mlp(ha, hb, W1, b1, W2, b2)


# speedup vs baseline: 2.1450x; 1.6161x over previous
import functools

import jax
import jax.numpy as jnp
from jax import lax
from jax.experimental import pallas as pl
from jax.experimental.pallas import tpu as pltpu
from jax.experimental.pallas import tpu_sc as plsc

N_NODES = 10000
N_EDGES = 160000
D_IN = 256
W_HID = 512
D_OUT = 256

NC = 2
NS = 16
DH = 128

K = 48
NB = 7
NCHUNK = 210
EPT = NCHUNK * K
E_PAD = EPT * NS
NPT = 624
NPT_LAST = N_NODES - (NS - 1) * NPT
ACC_ROWS = N_NODES + 16


def _sc_aggregate(xa, xb, src, dst):
    mesh = plsc.VectorSubcoreMesh(
        core_axis_name="c", subcore_axis_name="s", num_cores=NC, num_subcores=NS
    )

    @functools.partial(
        pl.kernel,
        out_type=(
            jax.ShapeDtypeStruct((N_NODES, DH), jnp.float32),
            jax.ShapeDtypeStruct((N_NODES, DH), jnp.float32),
        ),
        mesh=mesh,
        scratch_types=[
            [pltpu.VMEM((K,), jnp.int32) for _ in range(NB)],
            [pltpu.VMEM((K,), jnp.int32) for _ in range(NB)],
            [pltpu.VMEM((K, DH), jnp.float32) for _ in range(NB)],
            pltpu.VMEM_SHARED((ACC_ROWS, DH), jnp.float32),
            pltpu.SemaphoreType.DMA,
            pltpu.SemaphoreType.DMA,
            pltpu.SemaphoreType.DMA,
        ],
    )
    def body(xa_hbm, xb_hbm, src_hbm, dst_hbm, outa_hbm, outb_hbm,
             sidx, didx, rows, acc, sem_g, sem_si, sem_di):
        c = lax.axis_index("c")
        s = lax.axis_index("s")
        base = s * EPT

        def sidx_copy(j, b):
            off = pl.multiple_of(base + j * K, K)
            pltpu.async_copy(src_hbm.at[pl.ds(off, K)], sidx[b], sem_si)

        def didx_copy(j, b):
            off = pl.multiple_of(base + j * K, K)
            pltpu.async_copy(dst_hbm.at[pl.ds(off, K)], didx[b], sem_di)

        def wait_sidx(b):
            pltpu.make_async_copy(src_hbm.at[pl.ds(0, K)], sidx[b], sem_si).wait()

        def wait_didx(b):
            pltpu.make_async_copy(dst_hbm.at[pl.ds(0, K)], didx[b], sem_di).wait()

        def gather(x_hbm, b):
            return pltpu.async_copy(x_hbm.at[sidx[b]], rows[b], sem_g)

        def wait_gather(x_hbm, b):
            pltpu.make_async_copy(x_hbm.at[sidx[b]], rows[b], sem_g).wait()

        def prime(x_hbm):
            for b in range(NB):
                sidx_copy(b, b)
                didx_copy(b, b)
            for b in range(NB):
                wait_sidx(b)
                gather(x_hbm, b)

        @pl.when(c == 0)
        def _():
            prime(xa_hbm)

        @pl.when(c == 1)
        def _():
            prime(xb_hbm)

        def seed(x_hbm):
            @pl.when(s < NS - 1)
            def _():
                pltpu.sync_copy(x_hbm.at[pl.ds(s * NPT, NPT)],
                                acc.at[pl.ds(s * NPT, NPT)])

            @pl.when(s == NS - 1)
            def _():
                pltpu.sync_copy(x_hbm.at[pl.ds((NS - 1) * NPT, NPT_LAST)],
                                acc.at[pl.ds((NS - 1) * NPT, NPT_LAST)])

        @pl.when(c == 0)
        def _():
            seed(xa_hbm)

        @pl.when(c == 1)
        def _():
            seed(xb_hbm)

        plsc.subcore_barrier()

        def run(x_hbm):
            def step(j, b):
                wait_gather(x_hbm, b)

                @pl.when(j + NB < NCHUNK)
                def _():
                    sidx_copy(j + NB, b)

                wait_didx(b)
                pltpu.sync_copy(rows[b], acc.at[didx[b]], add=True)

                @pl.when(j + NB < NCHUNK)
                def _():
                    wait_sidx(b)
                    gather(x_hbm, b)
                    didx_copy(j + NB, b)

            def group(p, carry):
                j0 = NB * p
                for b in range(NB):
                    step(j0 + b, b)
                return carry

            lax.fori_loop(0, NCHUNK // NB, group, 0)

        @pl.when(c == 0)
        def _():
            run(xa_hbm)

        @pl.when(c == 1)
        def _():
            run(xb_hbm)

        plsc.subcore_barrier()

        def writeback(out_hbm):
            @pl.when(s < NS - 1)
            def _():
                pltpu.sync_copy(acc.at[pl.ds(s * NPT, NPT)],
                                out_hbm.at[pl.ds(s * NPT, NPT)])

            @pl.when(s == NS - 1)
            def _():
                pltpu.sync_copy(acc.at[pl.ds((NS - 1) * NPT, NPT_LAST)],
                                out_hbm.at[pl.ds((NS - 1) * NPT, NPT_LAST)])

        @pl.when(c == 0)
        def _():
            writeback(outa_hbm)

        @pl.when(c == 1)
        def _():
            writeback(outb_hbm)

    return body(xa, xb, src, dst)


def _mlp_body(ha_ref, hb_ref, w1_ref, b1_ref, w2_ref, b2_ref, o_ref):
    h = jnp.concatenate([ha_ref[...], hb_ref[...]], axis=1)
    z = jnp.dot(h, w1_ref[...], preferred_element_type=jnp.float32) + b1_ref[...]
    z = jnp.maximum(z, 0.0)
    o_ref[...] = (
        jnp.dot(z, w2_ref[...], preferred_element_type=jnp.float32) + b2_ref[...]
    )


def _mlp(ha, hb, W1, b1, W2, b2):
    BN = 1000
    grid = (N_NODES // BN,)
    return pl.pallas_call(
        _mlp_body,
        grid=grid,
        in_specs=[
            pl.BlockSpec((BN, 128), lambda i: (i, 0)),
            pl.BlockSpec((BN, 128), lambda i: (i, 0)),
            pl.BlockSpec((D_IN, W_HID), lambda i: (0, 0)),
            pl.BlockSpec((1, W_HID), lambda i: (0, 0)),
            pl.BlockSpec((W_HID, D_OUT), lambda i: (0, 0)),
            pl.BlockSpec((1, D_OUT), lambda i: (0, 0)),
        ],
        out_specs=pl.BlockSpec((BN, D_OUT), lambda i: (i, 0)),
        out_shape=jax.ShapeDtypeStruct((N_NODES, D_OUT), jnp.float32),
    )(ha, hb, W1, b1.reshape(1, W_HID), W2, b2.reshape(1, D_OUT))


def kernel(x, edge_index, W1, b1, W2, b2):
    src = edge_index[0].astype(jnp.int32)
    dst = edge_index[1].astype(jnp.int32)
    pad = E_PAD - N_EDGES
    src = jnp.concatenate([src, jnp.zeros((pad,), jnp.int32)])
    dst = jnp.concatenate([dst, jnp.full((pad,), N_NODES, jnp.int32)])
    xa = x[:, :DH]
    xb = x[:, DH:2*DH]
    ha, hb = _sc_aggregate(xa, xb, src, dst)
    return _mlp(ha, hb, W1, b1, W2, b2)
